# baseline (speedup 1.0000x reference)
import functools

import jax
import jax.numpy as jnp
from jax import lax
from jax.experimental import pallas as pl
from jax.experimental.pallas import tpu as pltpu
from jax.experimental.pallas import tpu_sc as plsc

_INFO = plsc.get_sparse_core_info()
_NC = _INFO.num_cores
_NS = _INFO.num_subcores
_NW = _NC * _NS

_CHUNK = 128


def kernel(input_ids, decoder_input_ids, table):
    B, S_enc = input_ids.shape
    _, S_dec = decoder_input_ids.shape
    V, D = table.shape
    n_enc = B * S_enc
    n_dec = B * S_dec
    enc_per_w = n_enc // _NW
    dec_per_w = n_dec // _NW
    k_enc = enc_per_w // _CHUNK
    k_dec = dec_per_w // _CHUNK
    k_tot = k_enc + k_dec
    wpr_enc = S_enc // enc_per_w
    wpr_dec = S_dec // dec_per_w

    mesh = plsc.VectorSubcoreMesh(core_axis_name="c", subcore_axis_name="s")

    @functools.partial(
        pl.kernel,
        mesh=mesh,
        out_type=(
            jax.ShapeDtypeStruct((B, S_enc, D), jnp.float32),
            jax.ShapeDtypeStruct((B, S_dec, D), jnp.float32),
        ),
        scratch_types=[
            pltpu.VMEM((k_tot * _CHUNK,), jnp.int32),
            pltpu.VMEM((k_tot * _CHUNK, D), jnp.float32),
            pltpu.SemaphoreType.DMA((k_tot,)),
            pltpu.SemaphoreType.DMA((k_tot,)),
            pltpu.SemaphoreType.DMA,
        ],
    )
    def k(enc_hbm, dec_hbm, table_hbm, out_enc, out_dec, idx_v, rows_v, isem, gsem, osem):
        wid = lax.axis_index("s") * _NC + lax.axis_index("c")

        def src_slice(j):
            if j < k_enc:
                return enc_hbm.at[
                    wid // wpr_enc,
                    pl.ds((wid % wpr_enc) * enc_per_w + j * _CHUNK, _CHUNK),
                ]
            return dec_hbm.at[
                wid // wpr_dec,
                pl.ds((wid % wpr_dec) * dec_per_w + (j - k_enc) * _CHUNK, _CHUNK),
            ]

        def dst_slice(j):
            if j < k_enc:
                flat = wid * enc_per_w + j * _CHUNK
                return out_enc.at[flat // S_enc, pl.ds(flat % S_enc, _CHUNK)]
            flat = wid * dec_per_w + (j - k_enc) * _CHUNK
            return out_dec.at[flat // S_dec, pl.ds(flat % S_dec, _CHUNK)]

        idx_copies = [
            pltpu.async_copy(src_slice(j), idx_v.at[pl.ds(j * _CHUNK, _CHUNK)], isem.at[j])
            for j in range(k_tot)
        ]
        gathers = []
        for j in range(k_tot):
            idx_copies[j].wait()
            gathers.append(
                pltpu.async_copy(
                    table_hbm.at[idx_v.at[pl.ds(j * _CHUNK, _CHUNK)]],
                    rows_v.at[pl.ds(j * _CHUNK, _CHUNK)],
                    gsem.at[j],
                )
            )
        outs = []
        for j in range(k_tot):
            gathers[j].wait()
            outs.append(
                pltpu.async_copy(rows_v.at[pl.ds(j * _CHUNK, _CHUNK)], dst_slice(j), osem)
            )
        for o in outs:
            o.wait()

    return k(input_ids, decoder_input_ids, table)
